# baseline (speedup 1.0000x reference)
import functools

import jax
import jax.numpy as jnp
from jax import lax
from jax.experimental import pallas as pl
from jax.experimental.pallas import tpu as pltpu
from jax.experimental.pallas import tpu_sc as plsc

NUM_EMB = 1000000
DIM = 32
L = 16

NC = 2
NS = 16
NW = NC * NS

B = 4096 * 200
B_PER_W = B // NW
CHUNK = 1024
N_CHUNKS = B_PER_W // CHUNK
G = 128
NG = CHUNK // G


def _body(idx_hbm, table_hbm, out_hbm, idx_v, rows_v, out_v, gsem):
    wid = lax.axis_index("s") * NC + lax.axis_index("c")
    base = wid * B_PER_W

    evens = lax.iota(jnp.int32, L) * 2
    odds = evens + 1

    def chunk_step(g, _):
        row0 = pl.multiple_of(base + g * CHUNK, CHUNK)
        pltpu.sync_copy(idx_hbm.at[pl.ds(pl.multiple_of(row0 // G, 8), NG)], idx_v)
        for j in range(NG):
            pltpu.async_copy(
                table_hbm.at[idx_v.at[j]],
                rows_v.at[pl.ds(j * G, G)],
                gsem,
            )
        pltpu.make_async_copy(
            table_hbm.at[idx_v.at[0]], rows_v.at[pl.ds(0, G)], gsem
        ).wait()
        for j in range(1, NG):
            pltpu.make_async_copy(
                table_hbm.at[idx_v.at[j]], rows_v.at[pl.ds(j * G, G)], gsem
            ).wait()

        def cvt(i, _):
            row = jnp.full((L,), i, dtype=jnp.int32)
            a = plsc.load_gather(rows_v, [row, evens])
            b = plsc.load_gather(rows_v, [row, odds])
            p = plsc.pack(a, b, format=plsc.PackFormat.INTERLEAVED)
            out_v[i, :] = p
            return 0

        lax.fori_loop(0, CHUNK, cvt, 0, unroll=8)
        pltpu.sync_copy(out_v, out_hbm.at[pl.ds(row0, CHUNK)])
        return 0

    lax.fori_loop(0, N_CHUNKS, chunk_step, 0)


@jax.jit
def _lookup(idx2d, table):
    mesh = plsc.VectorSubcoreMesh(core_axis_name="c", subcore_axis_name="s")
    return pl.kernel(
        _body,
        out_type=jax.ShapeDtypeStruct((B, DIM), jnp.bfloat16),
        mesh=mesh,
        scratch_types=[
            pltpu.VMEM((NG, G), jnp.int32),
            pltpu.VMEM((CHUNK, DIM), jnp.float32),
            pltpu.VMEM((CHUNK, DIM), jnp.bfloat16),
            pltpu.SemaphoreType.DMA,
        ],
        compiler_params=pltpu.CompilerParams(
            needs_layout_passes=False, use_tc_tiling_on_sc=False
        ),
    )(idx2d, table)


def kernel(input, embedding_weight):
    idx = input.reshape(B // G, G).astype(jnp.int32)
    out = _lookup(idx, embedding_weight)
    return out.reshape(input.shape + (DIM,))


# speedup vs baseline: 1.0519x; 1.0519x over previous
import jax
import jax.numpy as jnp
from jax import lax
from jax.experimental import pallas as pl
from jax.experimental.pallas import tpu as pltpu
from jax.experimental.pallas import tpu_sc as plsc

NUM_EMB = 1000000
DIM = 32
L = 16

NC = 2
NS = 16
NW = NC * NS

B = 4096 * 200
B_PER_W = B // NW
CHUNK = 1024
N_CHUNKS = B_PER_W // CHUNK
G = 128
NG = CHUNK // G


def _body(idx_hbm, table_hbm, out_hbm, idx_v, rows_v, out_v, gsem, osem):
    wid = lax.axis_index("s") * NC + lax.axis_index("c")
    base = wid * B_PER_W

    evens = lax.iota(jnp.int32, L) * 2
    odds = evens + 1

    def stage_and_fire(g, p):
        row0 = pl.multiple_of(base + g * CHUNK, CHUNK)
        pltpu.sync_copy(
            idx_hbm.at[pl.ds(pl.multiple_of(row0 // G, 8), NG)], idx_v.at[p]
        )
        for j in range(NG):
            pltpu.async_copy(
                table_hbm.at[idx_v.at[p, j]],
                rows_v.at[p, pl.ds(j * G, G)],
                gsem.at[p],
            )

    def drain_gathers(p):
        for j in range(NG):
            pltpu.make_async_copy(
                table_hbm.at[idx_v.at[p, j]],
                rows_v.at[p, pl.ds(j * G, G)],
                gsem.at[p],
            ).wait()

    def convert(p):
        rv = rows_v.at[p]
        ov = out_v.at[p]

        def cvt(i, _):
            row = jnp.full((L,), i, dtype=jnp.int32)
            a = plsc.load_gather(rv, [row, evens])
            b = plsc.load_gather(rv, [row, odds])
            ov[i, :] = plsc.pack(a, b, format=plsc.PackFormat.INTERLEAVED)
            return 0

        lax.fori_loop(0, CHUNK, cvt, 0, unroll=8)

    def out_copy(g, p):
        row0 = pl.multiple_of(base + g * CHUNK, CHUNK)
        return pltpu.make_async_copy(
            out_v.at[p], out_hbm.at[pl.ds(row0, CHUNK)], osem.at[p]
        )

    stage_and_fire(0, 0)
    for g in range(N_CHUNKS):
        p = g % 2
        if g + 1 < N_CHUNKS:
            stage_and_fire(g + 1, 1 - p)
        drain_gathers(p)
        if g >= 2:
            out_copy(g - 2, p).wait()
        convert(p)
        out_copy(g, p).start()
    out_copy(N_CHUNKS - 2, N_CHUNKS % 2).wait()
    out_copy(N_CHUNKS - 1, 1 - N_CHUNKS % 2).wait()


@jax.jit
def _lookup(idx2d, table):
    mesh = plsc.VectorSubcoreMesh(core_axis_name="c", subcore_axis_name="s")
    return pl.kernel(
        _body,
        out_type=jax.ShapeDtypeStruct((B, DIM), jnp.bfloat16),
        mesh=mesh,
        scratch_types=[
            pltpu.VMEM((2, NG, G), jnp.int32),
            pltpu.VMEM((2, CHUNK, DIM), jnp.float32),
            pltpu.VMEM((2, CHUNK, DIM), jnp.bfloat16),
            pltpu.SemaphoreType.DMA((2,)),
            pltpu.SemaphoreType.DMA((2,)),
        ],
        compiler_params=pltpu.CompilerParams(
            needs_layout_passes=False, use_tc_tiling_on_sc=False
        ),
    )(idx2d, table)


def kernel(input, embedding_weight):
    idx = input.reshape(B // G, G).astype(jnp.int32)
    out = _lookup(idx, embedding_weight)
    return out.reshape(input.shape + (DIM,))


# speedup vs baseline: 1.0535x; 1.0015x over previous
import jax
import jax.numpy as jnp
from jax import lax
from jax.experimental import pallas as pl
from jax.experimental.pallas import tpu as pltpu
from jax.experimental.pallas import tpu_sc as plsc

DIM = 32
L = 16
NC = 2
NS = 16
NW = NC * NS

N_ROWS = 4096
N_COLS = 200
ROWS_PER_W = N_ROWS // NW
R = 4
N_CHUNKS = ROWS_PER_W // R
CR = R * N_COLS


def _body(idx_hbm, table_hbm, out_hbm, idx_v, rows_v, out_v, gsem, osem):
    wid = lax.axis_index("s") * NC + lax.axis_index("c")
    base = wid * ROWS_PER_W

    evens = lax.iota(jnp.int32, L) * 2
    odds = evens + 1

    def stage_and_fire(g):
        p = g % 2
        row0 = base + g * R
        pltpu.sync_copy(idx_hbm.at[pl.ds(row0, R)], idx_v.at[p])
        for rr in range(R):
            pltpu.async_copy(
                table_hbm.at[idx_v.at[p, rr, pl.ds(0, 128)]],
                rows_v.at[p, pl.ds(rr * N_COLS, 128)],
                gsem.at[p],
            )
            pltpu.async_copy(
                table_hbm.at[idx_v.at[p, rr, pl.ds(128, 72)]],
                rows_v.at[p, pl.ds(rr * N_COLS + 128, 72)],
                gsem.at[p],
            )

    def drain_gathers(g):
        p = g % 2
        pltpu.make_async_copy(
            table_hbm.at[pl.ds(0, CR)],
            rows_v.at[p],
            gsem.at[p],
        ).wait()

    def convert(g):
        p = g % 2
        for rr in range(R):
            def cvt(i, _, rr=rr, p=p):
                row = jnp.full((L,), rr * N_COLS + i, dtype=jnp.int32)
                pv = jnp.full((L,), p, dtype=jnp.int32)
                a = plsc.load_gather(rows_v, [pv, row, evens])
                b = plsc.load_gather(rows_v, [pv, row, odds])
                out_v[p, rr, i, :] = plsc.pack(
                    a, b, format=plsc.PackFormat.INTERLEAVED
                )
                return 0

            lax.fori_loop(0, N_COLS, cvt, 0, unroll=8)

    def out_copy(g):
        p = g % 2
        row0 = base + g * R
        return pltpu.make_async_copy(
            out_v.at[p], out_hbm.at[pl.ds(row0, R)], osem.at[p]
        )

    def step(g, _):
        @pl.when(g + 1 < N_CHUNKS)
        def _():
            stage_and_fire(g + 1)

        drain_gathers(g)

        @pl.when(g >= 2)
        def _():
            out_copy(g - 2).wait()

        convert(g)
        out_copy(g).start()
        return 0

    stage_and_fire(0)
    lax.fori_loop(0, N_CHUNKS, step, 0)
    out_copy(N_CHUNKS - 2).wait()
    out_copy(N_CHUNKS - 1).wait()


@jax.jit
def _lookup(idx, table):
    mesh = plsc.VectorSubcoreMesh(core_axis_name="c", subcore_axis_name="s")
    return pl.kernel(
        _body,
        out_type=jax.ShapeDtypeStruct((N_ROWS, N_COLS, DIM), jnp.bfloat16),
        mesh=mesh,
        scratch_types=[
            pltpu.VMEM((2, R, N_COLS), jnp.int32),
            pltpu.VMEM((2, CR, DIM), jnp.float32),
            pltpu.VMEM((2, R, N_COLS, DIM), jnp.bfloat16),
            pltpu.SemaphoreType.DMA((2,)),
            pltpu.SemaphoreType.DMA((2,)),
        ],
        compiler_params=pltpu.CompilerParams(
            needs_layout_passes=False, use_tc_tiling_on_sc=False
        ),
    )(idx, table)


def kernel(input, embedding_weight):
    return _lookup(input.astype(jnp.int32), embedding_weight)


# speedup vs baseline: 1.0679x; 1.0137x over previous
import jax
import jax.numpy as jnp
from jax import lax
from jax.experimental import pallas as pl
from jax.experimental.pallas import tpu as pltpu
from jax.experimental.pallas import tpu_sc as plsc

DIM = 32
NC = 2
NS = 16
NW = NC * NS

N_ROWS = 4096
N_COLS = 200
ROWS_PER_W = N_ROWS // NW
R = 8
N_CHUNKS = ROWS_PER_W // R
CR = R * N_COLS


def _body(idx_hbm, table_hbm, out_hbm, idx_v, out_v, gsem, osem):
    wid = lax.axis_index("s") * NC + lax.axis_index("c")
    base = wid * ROWS_PER_W

    def stage_and_fire(g):
        p = g % 2
        row0 = base + g * R
        pltpu.sync_copy(idx_hbm.at[pl.ds(row0, R)], idx_v.at[p])
        for rr in range(R):
            pltpu.async_copy(
                table_hbm.at[idx_v.at[p, rr, pl.ds(0, 128)]],
                out_v.at[p, rr, pl.ds(0, 128)],
                gsem.at[p],
            )
            pltpu.async_copy(
                table_hbm.at[idx_v.at[p, rr, pl.ds(128, 72)]],
                out_v.at[p, rr, pl.ds(128, 72)],
                gsem.at[p],
            )

    def drain_gathers(g):
        p = g % 2
        pltpu.make_async_copy(
            table_hbm.at[pl.ds(0, CR)],
            out_v.at[p],
            gsem.at[p],
        ).wait()

    def out_copy(g):
        p = g % 2
        row0 = base + g * R
        return pltpu.make_async_copy(
            out_v.at[p], out_hbm.at[pl.ds(row0, R)], osem.at[p]
        )

    def step(g, _):
        @pl.when(g + 1 < N_CHUNKS)
        def _():
            stage_and_fire(g + 1)

        drain_gathers(g)

        @pl.when(g >= 2)
        def _():
            out_copy(g - 2).wait()

        out_copy(g).start()
        return 0

    stage_and_fire(0)
    lax.fori_loop(0, N_CHUNKS, step, 0)
    out_copy(N_CHUNKS - 2).wait()
    out_copy(N_CHUNKS - 1).wait()


@jax.jit
def _lookup(idx, table_bf16):
    mesh = plsc.VectorSubcoreMesh(core_axis_name="c", subcore_axis_name="s")
    return pl.kernel(
        _body,
        out_type=jax.ShapeDtypeStruct((N_ROWS, N_COLS, DIM), jnp.bfloat16),
        mesh=mesh,
        scratch_types=[
            pltpu.VMEM((2, R, N_COLS), jnp.int32),
            pltpu.VMEM((2, R, N_COLS, DIM), jnp.bfloat16),
            pltpu.SemaphoreType.DMA((2,)),
            pltpu.SemaphoreType.DMA((2,)),
        ],
        compiler_params=pltpu.CompilerParams(
            needs_layout_passes=False, use_tc_tiling_on_sc=False
        ),
    )(idx, table_bf16)


def kernel(input, embedding_weight):
    return _lookup(
        input.astype(jnp.int32), embedding_weight.astype(jnp.bfloat16)
    )
